# baseline (speedup 1.0000x reference)
import functools

import jax
import jax.numpy as jnp
from jax.experimental import pallas as pl
from jax.experimental.pallas import tpu as pltpu

_NUM_HEADS = 8
_CHUNK = 128


def _erf(x):
    a1, a2, a3, a4, a5 = (0.254829592, -0.284496736, 1.421413741,
                          -1.453152027, 1.061405429)
    p = 0.3275911
    s = jnp.sign(x)
    ax = jnp.abs(x)
    t = 1.0 / (1.0 + p * ax)
    poly = ((((a5 * t + a4) * t + a3) * t + a2) * t + a1) * t
    return s * (1.0 - poly * jnp.exp(-ax * ax))


def _gelu_exact(x):
    return 0.5 * x * (1.0 + _erf(x * 0.7071067811865476))


def _proj(x, w1t, b1, g, beta, w2t, b2):
    h = jnp.dot(x, w1t, preferred_element_type=jnp.float32) + b1
    mu = jnp.mean(h, axis=-1, keepdims=True)
    var = jnp.mean((h - mu) ** 2, axis=-1, keepdims=True)
    h = (h - mu) * jax.lax.rsqrt(var + 1e-5) * g + beta
    h = _gelu_exact(h)
    return jnp.dot(h, w2t, preferred_element_type=jnp.float32) + b2


def _retriever_body(qf_ref, cf_ref, temp_ref,
                    qw1t_ref, qb1_ref, qg_ref, qbeta_ref, qw2t_ref, qb2_ref,
                    kw1t_ref, kb1_ref, kg_ref, kbeta_ref, kw2t_ref, kb2_ref,
                    wqt_ref, bq_ref, wkt_ref, bk_ref,
                    fw1_ref, fb1_ref, fw2_ref, fb2_ref,
                    out_ref, kp_s, katt_s, kn2r_s, invknr_s):
    f32 = jnp.float32
    qp = _proj(qf_ref[:], qw1t_ref[:], qb1_ref[:], qg_ref[:], qbeta_ref[:],
               qw2t_ref[:], qb2_ref[:])
    kp = _proj(cf_ref[:], kw1t_ref[:], kb1_ref[:], kg_ref[:], kbeta_ref[:],
               kw2t_ref[:], kb2_ref[:])

    qn2 = jnp.sum(qp * qp, axis=1, keepdims=True)
    kn2 = jnp.sum(kp * kp, axis=1, keepdims=True)
    inv_qn = 1.0 / jnp.maximum(jnp.sqrt(qn2), 1e-12)

    q_att = jnp.dot(qp, wqt_ref[:], preferred_element_type=f32) + bq_ref[:]
    k_att = jnp.dot(kp, wkt_ref[:], preferred_element_type=f32) + bk_ref[:]
    dh = q_att.shape[1] // _NUM_HEADS
    scale = 1.0 / (dh ** 0.5)
    n = kp.shape[0]
    n_chunks = n // _CHUNK

    kp_s[:] = kp
    katt_s[:] = k_att
    kn2r_s[:] = kn2.T
    invknr_s[:] = (1.0 / jnp.maximum(jnp.sqrt(kn2), 1e-12)).T

    dot_t = lambda a, b: jax.lax.dot_general(
        a, b, (((1,), (1,)), ((), ())), preferred_element_type=f32)

    def head_exp(i, h):
        kc = katt_s[pl.ds(i * _CHUNK, _CHUNK), h * dh:(h + 1) * dh]
        return jnp.exp(dot_t(q_att[:, h * dh:(h + 1) * dh], kc) * scale)

    def denom_body(i, sums):
        return tuple(
            sums[h] + jnp.sum(head_exp(i, h), axis=1, keepdims=True)
            for h in range(_NUM_HEADS))

    zero = jnp.zeros((qp.shape[0], 1), f32)
    sums = jax.lax.fori_loop(0, n_chunks, denom_body, (zero,) * _NUM_HEADS)
    rinv = tuple((1.0 / _NUM_HEADS) / s for s in sums)

    temp = temp_ref[0, 0]
    fb2 = fb2_ref[0, 0]

    def fuse_body(i, carry):
        kpc = kp_s[pl.ds(i * _CHUNK, _CHUNK), :]
        kn2c = kn2r_s[:, pl.ds(i * _CHUNK, _CHUNK)]
        inv_knc = invknr_s[:, pl.ds(i * _CHUNK, _CHUNK)]
        g = dot_t(qp, kpc)
        cos = g * (inv_qn * temp) * inv_knc
        d2 = jnp.maximum(qn2 + kn2c - 2.0 * g, 0.0)
        eu = 1.0 / (1.0 + jnp.sqrt(d2))
        learned = rinv[0] * head_exp(i, 0)
        for h in range(1, _NUM_HEADS):
            learned = learned + rinv[h] * head_exp(i, h)
        acc = jnp.zeros_like(g)
        for j in range(fw1_ref.shape[0]):
            t = (cos * fw1_ref[j, 0] + eu * fw1_ref[j, 1]
                 + learned * fw1_ref[j, 2] + fb1_ref[0, j])
            acc = acc + jnp.maximum(t, 0.0) * fw2_ref[0, j]
        out_ref[:, pl.ds(i * _CHUNK, _CHUNK)] = jax.nn.sigmoid(acc + fb2)
        return carry

    jax.lax.fori_loop(0, n_chunks, fuse_body, 0)


@jax.jit
def kernel(query_features, candidate_features, log_temp,
           qp_w1, qp_b1, qp_ln_g, qp_ln_b, qp_w2, qp_b2,
           kp_w1, kp_b1, kp_ln_g, kp_ln_b, kp_w2, kp_b2,
           attn_wq, attn_bq, attn_wk, attn_bk,
           fus_w1, fus_b1, fus_w2, fus_b2):
    b, d = query_features.shape
    n = candidate_features.shape[0]
    f32 = jnp.float32
    row = lambda v: v.reshape(1, -1).astype(f32)

    temp = jnp.exp(log_temp).reshape(1, 1).astype(f32)
    args = (
        query_features.astype(f32), candidate_features.astype(f32), temp,
        qp_w1.T.astype(f32), row(qp_b1), row(qp_ln_g), row(qp_ln_b),
        qp_w2.T.astype(f32), row(qp_b2),
        kp_w1.T.astype(f32), row(kp_b1), row(kp_ln_g), row(kp_ln_b),
        kp_w2.T.astype(f32), row(kp_b2),
        attn_wq.T.astype(f32), row(attn_bq),
        attn_wk.T.astype(f32), row(attn_bk),
        fus_w1.astype(f32), row(fus_b1), fus_w2.reshape(1, -1).astype(f32),
        fus_b2.reshape(1, 1).astype(f32),
    )

    vmem = pl.BlockSpec(memory_space=pltpu.VMEM)
    smem = pl.BlockSpec(memory_space=pltpu.SMEM)
    in_specs = [vmem, vmem, smem] + [vmem] * 16 + [smem] * 4

    return pl.pallas_call(
        _retriever_body,
        out_shape=jax.ShapeDtypeStruct((b, n), f32),
        in_specs=in_specs,
        out_specs=vmem,
        scratch_shapes=[
            pltpu.VMEM((n, d), f32), pltpu.VMEM((n, d), f32),
            pltpu.VMEM((1, n), f32), pltpu.VMEM((1, n), f32),
        ],
    )(*args)


# speedup vs baseline: 1.2268x; 1.2268x over previous
import functools

import jax
import jax.numpy as jnp
from jax.experimental import pallas as pl
from jax.experimental.pallas import tpu as pltpu

_NUM_HEADS = 8
_CHUNK = 128


def _erf(x):
    a1, a2, a3, a4, a5 = (0.254829592, -0.284496736, 1.421413741,
                          -1.453152027, 1.061405429)
    p = 0.3275911
    s = jnp.sign(x)
    ax = jnp.abs(x)
    t = 1.0 / (1.0 + p * ax)
    poly = ((((a5 * t + a4) * t + a3) * t + a2) * t + a1) * t
    return s * (1.0 - poly * jnp.exp(-ax * ax))


def _gelu_exact(x):
    return 0.5 * x * (1.0 + _erf(x * 0.7071067811865476))


def _proj(x, w1t, b1, g, beta, w2t, b2):
    h = jnp.dot(x, w1t, preferred_element_type=jnp.float32) + b1
    mu = jnp.mean(h, axis=-1, keepdims=True)
    var = jnp.mean((h - mu) ** 2, axis=-1, keepdims=True)
    h = (h - mu) * jax.lax.rsqrt(var + 1e-5) * g + beta
    h = _gelu_exact(h)
    return jnp.dot(h, w2t, preferred_element_type=jnp.float32) + b2


def _retriever_body(qf_ref, cf_ref, temp_ref,
                    qw1t_ref, qb1_ref, qg_ref, qbeta_ref, qw2t_ref, qb2_ref,
                    kw1t_ref, kb1_ref, kg_ref, kbeta_ref, kw2t_ref, kb2_ref,
                    wqt_ref, bq_ref, wkt_ref, bk_ref,
                    fw1_ref, fb1_ref, fw2_ref, fb2_ref,
                    out_ref):
    f32 = jnp.float32
    qp = _proj(qf_ref[:], qw1t_ref[:], qb1_ref[:], qg_ref[:], qbeta_ref[:],
               qw2t_ref[:], qb2_ref[:])
    kp = _proj(cf_ref[:], kw1t_ref[:], kb1_ref[:], kg_ref[:], kbeta_ref[:],
               kw2t_ref[:], kb2_ref[:])

    qn2 = jnp.sum(qp * qp, axis=1, keepdims=True)
    kn2r = jnp.sum(kp * kp, axis=1, keepdims=True).T
    inv_qn = 1.0 / jnp.maximum(jnp.sqrt(qn2), 1e-12)
    inv_knr = 1.0 / jnp.maximum(jnp.sqrt(kn2r), 1e-12)

    q_att = jnp.dot(qp, wqt_ref[:], preferred_element_type=f32) + bq_ref[:]
    k_att = jnp.dot(kp, wkt_ref[:], preferred_element_type=f32) + bk_ref[:]
    dh = q_att.shape[1] // _NUM_HEADS
    scale = 1.0 / (dh ** 0.5)
    n = kp.shape[0]
    n_chunks = n // _CHUNK

    dot_t = lambda a, b: jax.lax.dot_general(
        a, b, (((1,), (1,)), ((), ())), preferred_element_type=f32)

    def head_exp(i, h):
        kc = k_att[i * _CHUNK:(i + 1) * _CHUNK, h * dh:(h + 1) * dh]
        return jnp.exp(dot_t(q_att[:, h * dh:(h + 1) * dh], kc) * scale)

    sums = [jnp.zeros((qp.shape[0], 1), f32)] * _NUM_HEADS
    for i in range(n_chunks):
        for h in range(_NUM_HEADS):
            sums[h] = sums[h] + jnp.sum(head_exp(i, h), axis=1, keepdims=True)
    rinv = [(1.0 / _NUM_HEADS) / s for s in sums]

    temp = temp_ref[0, 0]
    fb2 = fb2_ref[0, 0]
    n_hidden = fw1_ref.shape[0]

    for i in range(n_chunks):
        sl = slice(i * _CHUNK, (i + 1) * _CHUNK)
        g = dot_t(qp, kp[sl, :])
        cos = g * (inv_qn * temp) * inv_knr[:, sl]
        d2 = jnp.maximum(qn2 + kn2r[:, sl] - 2.0 * g, 0.0)
        eu = 1.0 / (1.0 + jnp.sqrt(d2))
        learned = rinv[0] * head_exp(i, 0)
        for h in range(1, _NUM_HEADS):
            learned = learned + rinv[h] * head_exp(i, h)
        acc = jnp.zeros_like(g)
        for j in range(n_hidden):
            t = (cos * fw1_ref[j, 0] + eu * fw1_ref[j, 1]
                 + learned * fw1_ref[j, 2] + fb1_ref[0, j])
            acc = acc + jnp.maximum(t, 0.0) * fw2_ref[0, j]
        out_ref[:, sl] = jax.nn.sigmoid(acc + fb2)


@jax.jit
def kernel(query_features, candidate_features, log_temp,
           qp_w1, qp_b1, qp_ln_g, qp_ln_b, qp_w2, qp_b2,
           kp_w1, kp_b1, kp_ln_g, kp_ln_b, kp_w2, kp_b2,
           attn_wq, attn_bq, attn_wk, attn_bk,
           fus_w1, fus_b1, fus_w2, fus_b2):
    b, d = query_features.shape
    n = candidate_features.shape[0]
    f32 = jnp.float32
    row = lambda v: v.reshape(1, -1).astype(f32)

    temp = jnp.exp(log_temp).reshape(1, 1).astype(f32)
    args = (
        query_features.astype(f32), candidate_features.astype(f32), temp,
        qp_w1.T.astype(f32), row(qp_b1), row(qp_ln_g), row(qp_ln_b),
        qp_w2.T.astype(f32), row(qp_b2),
        kp_w1.T.astype(f32), row(kp_b1), row(kp_ln_g), row(kp_ln_b),
        kp_w2.T.astype(f32), row(kp_b2),
        attn_wq.T.astype(f32), row(attn_bq),
        attn_wk.T.astype(f32), row(attn_bk),
        fus_w1.astype(f32), row(fus_b1), fus_w2.reshape(1, -1).astype(f32),
        fus_b2.reshape(1, 1).astype(f32),
    )

    vmem = pl.BlockSpec(memory_space=pltpu.VMEM)
    smem = pl.BlockSpec(memory_space=pltpu.SMEM)
    in_specs = [vmem, vmem, smem] + [vmem] * 16 + [smem] * 4

    return pl.pallas_call(
        _retriever_body,
        out_shape=jax.ShapeDtypeStruct((b, n), f32),
        in_specs=in_specs,
        out_specs=vmem,
    )(*args)


# speedup vs baseline: 1.2835x; 1.0462x over previous
import functools

import jax
import jax.numpy as jnp
from jax.experimental import pallas as pl
from jax.experimental.pallas import tpu as pltpu

_NUM_HEADS = 8
_CHUNK = 256


def _erf(x):
    a1, a2, a3, a4, a5 = (0.254829592, -0.284496736, 1.421413741,
                          -1.453152027, 1.061405429)
    p = 0.3275911
    s = jnp.sign(x)
    ax = jnp.abs(x)
    t = 1.0 / (1.0 + p * ax)
    poly = ((((a5 * t + a4) * t + a3) * t + a2) * t + a1) * t
    return s * (1.0 - poly * jnp.exp(-ax * ax))


def _gelu_exact(x):
    return 0.5 * x * (1.0 + _erf(x * 0.7071067811865476))


def _proj(x, w1t, b1, g, beta, w2t, b2):
    h = jnp.dot(x, w1t, preferred_element_type=jnp.float32) + b1
    mu = jnp.mean(h, axis=-1, keepdims=True)
    var = jnp.mean((h - mu) ** 2, axis=-1, keepdims=True)
    h = (h - mu) * jax.lax.rsqrt(var + 1e-5) * g + beta
    h = _gelu_exact(h)
    return jnp.dot(h, w2t, preferred_element_type=jnp.float32) + b2


def _retriever_body(qf_ref, cf_ref, temp_ref,
                    qw1t_ref, qb1_ref, qg_ref, qbeta_ref, qw2t_ref, qb2_ref,
                    kw1t_ref, kb1_ref, kg_ref, kbeta_ref, kw2t_ref, kb2_ref,
                    wqt_ref, bq_ref, wkt_ref, bk_ref,
                    fw1_ref, fb1_ref, fw2_ref, fb2_ref,
                    out_ref):
    f32 = jnp.float32
    qp = _proj(qf_ref[:], qw1t_ref[:], qb1_ref[:], qg_ref[:], qbeta_ref[:],
               qw2t_ref[:], qb2_ref[:])
    kp = _proj(cf_ref[:], kw1t_ref[:], kb1_ref[:], kg_ref[:], kbeta_ref[:],
               kw2t_ref[:], kb2_ref[:])

    qn2 = jnp.sum(qp * qp, axis=1, keepdims=True)
    kn2r = jnp.sum(kp * kp, axis=1, keepdims=True).T
    inv_qn = 1.0 / jnp.maximum(jnp.sqrt(qn2), 1e-12)
    inv_knr = 1.0 / jnp.maximum(jnp.sqrt(kn2r), 1e-12)

    q_att = jnp.dot(qp, wqt_ref[:], preferred_element_type=f32) + bq_ref[:]
    k_att = jnp.dot(kp, wkt_ref[:], preferred_element_type=f32) + bk_ref[:]
    dh = q_att.shape[1] // _NUM_HEADS
    scale = 1.0 / (dh ** 0.5)
    n = kp.shape[0]
    n_chunks = n // _CHUNK

    dot_t = lambda a, b: jax.lax.dot_general(
        a, b, (((1,), (1,)), ((), ())), preferred_element_type=f32)

    def head_exp(i, h):
        kc = k_att[i * _CHUNK:(i + 1) * _CHUNK, h * dh:(h + 1) * dh]
        return jnp.exp(dot_t(q_att[:, h * dh:(h + 1) * dh], kc) * scale)

    sums = [jnp.zeros((qp.shape[0], 1), f32)] * _NUM_HEADS
    for i in range(n_chunks):
        for h in range(_NUM_HEADS):
            sums[h] = sums[h] + jnp.sum(head_exp(i, h), axis=1, keepdims=True)
    rinv = [(1.0 / _NUM_HEADS) / s for s in sums]

    temp = temp_ref[0, 0]
    fb2 = fb2_ref[0, 0]
    n_hidden = fw1_ref.shape[0]

    for i in range(n_chunks):
        sl = slice(i * _CHUNK, (i + 1) * _CHUNK)
        g = dot_t(qp, kp[sl, :])
        cos = g * (inv_qn * temp) * inv_knr[:, sl]
        d2 = jnp.maximum(qn2 + kn2r[:, sl] - 2.0 * g, 0.0)
        eu = 1.0 / (1.0 + jnp.sqrt(d2))
        learned = rinv[0] * head_exp(i, 0)
        for h in range(1, _NUM_HEADS):
            learned = learned + rinv[h] * head_exp(i, h)
        acc = jnp.zeros_like(g)
        for j in range(n_hidden):
            t = (cos * fw1_ref[j, 0] + eu * fw1_ref[j, 1]
                 + learned * fw1_ref[j, 2] + fb1_ref[0, j])
            acc = acc + jnp.maximum(t, 0.0) * fw2_ref[0, j]
        out_ref[:, sl] = jax.nn.sigmoid(acc + fb2)


@jax.jit
def kernel(query_features, candidate_features, log_temp,
           qp_w1, qp_b1, qp_ln_g, qp_ln_b, qp_w2, qp_b2,
           kp_w1, kp_b1, kp_ln_g, kp_ln_b, kp_w2, kp_b2,
           attn_wq, attn_bq, attn_wk, attn_bk,
           fus_w1, fus_b1, fus_w2, fus_b2):
    b, d = query_features.shape
    n = candidate_features.shape[0]
    f32 = jnp.float32
    row = lambda v: v.reshape(1, -1).astype(f32)

    temp = jnp.exp(log_temp).reshape(1, 1).astype(f32)
    args = (
        query_features.astype(f32), candidate_features.astype(f32), temp,
        qp_w1.T.astype(f32), row(qp_b1), row(qp_ln_g), row(qp_ln_b),
        qp_w2.T.astype(f32), row(qp_b2),
        kp_w1.T.astype(f32), row(kp_b1), row(kp_ln_g), row(kp_ln_b),
        kp_w2.T.astype(f32), row(kp_b2),
        attn_wq.T.astype(f32), row(attn_bq),
        attn_wk.T.astype(f32), row(attn_bk),
        fus_w1.astype(f32), row(fus_b1), fus_w2.reshape(1, -1).astype(f32),
        fus_b2.reshape(1, 1).astype(f32),
    )

    vmem = pl.BlockSpec(memory_space=pltpu.VMEM)
    smem = pl.BlockSpec(memory_space=pltpu.SMEM)
    in_specs = [vmem, vmem, smem] + [vmem] * 16 + [smem] * 4

    return pl.pallas_call(
        _retriever_body,
        out_shape=jax.ShapeDtypeStruct((b, n), f32),
        in_specs=in_specs,
        out_specs=vmem,
    )(*args)
